# baseline (speedup 1.0000x reference)
import functools

import jax
import jax.numpy as jnp
from jax import lax
from jax.experimental import pallas as pl
from jax.experimental.pallas import tpu as pltpu
from jax.experimental.pallas import tpu_sc as plsc

_LAMB = max(5.0, 1500.0 / 1.1)
_INV = 1.0 / (1.0 + _LAMB)
_LANES = 16
_WIN = 128
_CBLK = 2000


def _sc_gather(cosT, phiT, t32, n_rows, n_cols):
    info = plsc.get_sparse_core_info()
    nw = info.num_cores * info.num_subcores
    bpw = n_rows // nw
    mesh = plsc.VectorSubcoreMesh(core_axis_name="c", subcore_axis_name="s")

    @functools.partial(
        pl.kernel,
        mesh=mesh,
        out_type=(
            jax.ShapeDtypeStruct((n_rows, 8, _WIN), jnp.float32),
            jax.ShapeDtypeStruct((n_rows, 8, _WIN), jnp.float32),
        ),
        scratch_types=[
            pltpu.VMEM((bpw,), jnp.int32),
            pltpu.SemaphoreType.DMA,
        ],
    )
    def gather_kernel(cos_hbm, phi_hbm, t_hbm, cwin_out, pwin_out, t_v, sem):
        wid = lax.axis_index("s") * info.num_cores + lax.axis_index("c")
        base = wid * bpw
        pltpu.sync_copy(t_hbm.at[pl.ds(base, bpw)], t_v)
        copies = []
        for g in range(0, bpw, _LANES):
            c0v = t_v[pl.ds(g, _LANES)] & ~7
            for j in range(_LANES):
                r = base + g + j
                c0 = pl.multiple_of(c0v[j], 8)
                lc = pl.multiple_of((r // _WIN) * _WIN, _WIN)
                for src, dst in ((cos_hbm, cwin_out), (phi_hbm, pwin_out)):
                    copies.append(pltpu.async_copy(
                        src.at[pl.ds(c0, 8), pl.ds(lc, _WIN)],
                        dst.at[r], sem))
        for cp in copies:
            cp.wait()

    return gather_kernel(cosT, phiT, t32)


def _stream_body(cosT_ref, m_ref, s_ref, amax_ref):
    i = pl.program_id(0)
    x = cosT_ref[...]
    bm = jnp.max(x, axis=0, keepdims=True)
    rows = lax.broadcasted_iota(jnp.int32, x.shape, 0)
    bidx = jnp.min(jnp.where(x == bm, rows, jnp.int32(2**30)),
                   axis=0, keepdims=True) + i * _CBLK

    @pl.when(i == 0)
    def _init():
        m_ref[...] = jnp.full(bm.shape, -jnp.inf, jnp.float32)
        s_ref[...] = jnp.zeros(bm.shape, jnp.float32)
        amax_ref[...] = jnp.zeros(bm.shape, jnp.int32)

    m_old = m_ref[...]
    m_new = jnp.maximum(m_old, bm)
    bs = jnp.sum(jnp.exp(x - m_new), axis=0, keepdims=True)
    s_ref[...] = s_ref[...] * jnp.exp(m_old - m_new) + bs
    amax_ref[...] = jnp.where(bm > m_old, bidx, amax_ref[...])
    m_ref[...] = m_new


def _tc_stream(cosT):
    n_cols, n_rows = cosT.shape
    grid = (n_cols // _CBLK,)
    return pl.pallas_call(
        _stream_body,
        grid=grid,
        in_specs=[pl.BlockSpec((_CBLK, n_rows), lambda i: (i, 0))],
        out_specs=(
            pl.BlockSpec((1, n_rows), lambda i: (0, 0)),
            pl.BlockSpec((1, n_rows), lambda i: (0, 0)),
            pl.BlockSpec((1, n_rows), lambda i: (0, 0)),
        ),
        out_shape=(
            jax.ShapeDtypeStruct((1, n_rows), jnp.float32),
            jax.ShapeDtypeStruct((1, n_rows), jnp.float32),
            jax.ShapeDtypeStruct((1, n_rows), jnp.int32),
        ),
        compiler_params=pltpu.CompilerParams(
            dimension_semantics=("arbitrary",),
        ),
    )(cosT)


def _epilogue_body(m_ref, s_ref, amax_ref, trow_ref, tcol_ref,
                   cwin_ref, pwin_ref, loss_ref, acc_ref):
    bm = m_ref[...]
    s = s_ref[...]
    amax = amax_ref[...]
    trow = trow_ref[...]
    tcol = tcol_ref[...]
    n = tcol.shape[0]
    row3 = lax.broadcasted_iota(jnp.int32, (n, 8, _WIN), 0)
    sub3 = lax.broadcasted_iota(jnp.int32, (n, 8, _WIN), 1)
    lane3 = lax.broadcasted_iota(jnp.int32, (n, 8, _WIN), 2)
    fine = (sub3 == (tcol & 7)[:, :, None]) & (lane3 == (row3 & (_WIN - 1)))
    ct = jnp.sum(jnp.where(fine, cwin_ref[...], 0.0), axis=(1, 2))[None, :]
    pt = jnp.sum(jnp.where(fine, pwin_ref[...], 0.0), axis=(1, 2))[None, :]
    v = ct + (pt - ct) * _INV
    s_adj = s - jnp.exp(ct - bm) + jnp.exp(v - bm)
    logpt = v - bm - jnp.log(s_adj)
    scale = jnp.float32(1.0 / n)
    loss_ref[...] = -jnp.sum(logpt, axis=(0, 1), keepdims=True) * scale
    acc_ref[...] = jnp.sum((amax == trow).astype(jnp.float32),
                           axis=(0, 1), keepdims=True) * scale


def _tc_epilogue(m, s, amax, trow, tcol, cwin, pwin):
    n_rows = tcol.shape[0]
    return pl.pallas_call(
        _epilogue_body,
        in_specs=[
            pl.BlockSpec((1, n_rows), lambda: (0, 0)),
            pl.BlockSpec((1, n_rows), lambda: (0, 0)),
            pl.BlockSpec((1, n_rows), lambda: (0, 0)),
            pl.BlockSpec((1, n_rows), lambda: (0, 0)),
            pl.BlockSpec((n_rows, 1), lambda: (0, 0)),
            pl.BlockSpec((n_rows, 8, _WIN), lambda: (0, 0, 0)),
            pl.BlockSpec((n_rows, 8, _WIN), lambda: (0, 0, 0)),
        ],
        out_specs=(
            pl.BlockSpec((1, 1), lambda: (0, 0)),
            pl.BlockSpec((1, 1), lambda: (0, 0)),
        ),
        out_shape=(
            jax.ShapeDtypeStruct((1, 1), jnp.float32),
            jax.ShapeDtypeStruct((1, 1), jnp.float32),
        ),
    )(m, s, amax, trow, tcol, cwin, pwin)


def kernel(cos_theta, phi_theta, target):
    n_rows, n_cols = cos_theta.shape
    t32 = target.reshape(-1).astype(jnp.int32)
    cosT = cos_theta.T
    phiT = phi_theta.T
    cwin, pwin = _sc_gather(cosT, phiT, t32, n_rows, n_cols)
    m, s, amax = _tc_stream(cosT)
    loss, acc = _tc_epilogue(m, s, amax, t32.reshape(1, n_rows),
                             t32.reshape(n_rows, 1), cwin, pwin)
    return loss[0, 0], acc[0, 0]


# speedup vs baseline: 1.5111x; 1.5111x over previous
import functools

import jax
import jax.numpy as jnp
from jax import lax
from jax.experimental import pallas as pl
from jax.experimental.pallas import tpu as pltpu
from jax.experimental.pallas import tpu_sc as plsc

_LAMB = max(5.0, 1500.0 / 1.1)
_INV = 1.0 / (1.0 + _LAMB)
_LANES = 16
_WIN = 128
_CBLK = 2000


def _sc_gather(cosT, phiT, t32, n_rows, n_cols):
    info = plsc.get_sparse_core_info()
    nw = info.num_cores * info.num_subcores
    bpw = n_rows // nw
    mesh = plsc.VectorSubcoreMesh(core_axis_name="c", subcore_axis_name="s")

    @functools.partial(
        pl.kernel,
        mesh=mesh,
        out_type=(
            jax.ShapeDtypeStruct((n_rows, n_rows), jnp.float32),
            jax.ShapeDtypeStruct((n_rows, n_rows), jnp.float32),
        ),
        scratch_types=[
            pltpu.VMEM((bpw,), jnp.int32),
            pltpu.VMEM((bpw, n_rows), jnp.float32),
            pltpu.SemaphoreType.DMA,
        ],
    )
    def gather_kernel(cos_hbm, phi_hbm, t_hbm, cwin_out, pwin_out,
                      t_v, rows_v, sem):
        wid = lax.axis_index("s") * info.num_cores + lax.axis_index("c")
        base = wid * bpw
        pltpu.sync_copy(t_hbm.at[pl.ds(base, bpw)], t_v)
        for src, dst in ((cos_hbm, cwin_out), (phi_hbm, pwin_out)):
            pltpu.async_copy(src.at[t_v], rows_v, sem).wait()
            pltpu.sync_copy(rows_v, dst.at[pl.ds(base, bpw)])

    return gather_kernel(cosT, phiT, t32)


def _stream_body(cosT_ref, m_ref, s_ref, amax_ref):
    i = pl.program_id(0)
    x = cosT_ref[...]
    bm = jnp.max(x, axis=0, keepdims=True)
    rows = lax.broadcasted_iota(jnp.int32, x.shape, 0)
    bidx = jnp.min(jnp.where(x == bm, rows, jnp.int32(2**30)),
                   axis=0, keepdims=True) + i * _CBLK

    @pl.when(i == 0)
    def _init():
        m_ref[...] = jnp.full(bm.shape, -jnp.inf, jnp.float32)
        s_ref[...] = jnp.zeros(bm.shape, jnp.float32)
        amax_ref[...] = jnp.zeros(bm.shape, jnp.int32)

    m_old = m_ref[...]
    m_new = jnp.maximum(m_old, bm)
    bs = jnp.sum(jnp.exp(x - m_new), axis=0, keepdims=True)
    s_ref[...] = s_ref[...] * jnp.exp(m_old - m_new) + bs
    amax_ref[...] = jnp.where(bm > m_old, bidx, amax_ref[...])
    m_ref[...] = m_new


def _tc_stream(cosT):
    n_cols, n_rows = cosT.shape
    grid = (n_cols // _CBLK,)
    return pl.pallas_call(
        _stream_body,
        grid=grid,
        in_specs=[pl.BlockSpec((_CBLK, n_rows), lambda i: (i, 0))],
        out_specs=(
            pl.BlockSpec((1, n_rows), lambda i: (0, 0)),
            pl.BlockSpec((1, n_rows), lambda i: (0, 0)),
            pl.BlockSpec((1, n_rows), lambda i: (0, 0)),
        ),
        out_shape=(
            jax.ShapeDtypeStruct((1, n_rows), jnp.float32),
            jax.ShapeDtypeStruct((1, n_rows), jnp.float32),
            jax.ShapeDtypeStruct((1, n_rows), jnp.int32),
        ),
        compiler_params=pltpu.CompilerParams(
            dimension_semantics=("arbitrary",),
        ),
    )(cosT)


def _epilogue_body(m_ref, s_ref, amax_ref, trow_ref,
                   cwin_ref, pwin_ref, loss_ref, acc_ref):
    bm = m_ref[...]
    s = s_ref[...]
    amax = amax_ref[...]
    trow = trow_ref[...]
    n = trow.shape[1]
    row2 = lax.broadcasted_iota(jnp.int32, (n, n), 0)
    lane2 = lax.broadcasted_iota(jnp.int32, (n, n), 1)
    diag = row2 == lane2
    ct = jnp.sum(jnp.where(diag, cwin_ref[...], 0.0), axis=0, keepdims=True)
    pt = jnp.sum(jnp.where(diag, pwin_ref[...], 0.0), axis=0, keepdims=True)
    v = ct + (pt - ct) * _INV
    s_adj = s - jnp.exp(ct - bm) + jnp.exp(v - bm)
    logpt = v - bm - jnp.log(s_adj)
    scale = jnp.float32(1.0 / n)
    loss_ref[...] = -jnp.sum(logpt, axis=(0, 1), keepdims=True) * scale
    acc_ref[...] = jnp.sum((amax == trow).astype(jnp.float32),
                           axis=(0, 1), keepdims=True) * scale


def _tc_epilogue(m, s, amax, trow, cwin, pwin):
    n_rows = trow.shape[1]
    return pl.pallas_call(
        _epilogue_body,
        in_specs=[
            pl.BlockSpec((1, n_rows), lambda: (0, 0)),
            pl.BlockSpec((1, n_rows), lambda: (0, 0)),
            pl.BlockSpec((1, n_rows), lambda: (0, 0)),
            pl.BlockSpec((1, n_rows), lambda: (0, 0)),
            pl.BlockSpec((n_rows, n_rows), lambda: (0, 0)),
            pl.BlockSpec((n_rows, n_rows), lambda: (0, 0)),
        ],
        out_specs=(
            pl.BlockSpec((1, 1), lambda: (0, 0)),
            pl.BlockSpec((1, 1), lambda: (0, 0)),
        ),
        out_shape=(
            jax.ShapeDtypeStruct((1, 1), jnp.float32),
            jax.ShapeDtypeStruct((1, 1), jnp.float32),
        ),
    )(m, s, amax, trow, cwin, pwin)


def kernel(cos_theta, phi_theta, target):
    n_rows, n_cols = cos_theta.shape
    t32 = target.reshape(-1).astype(jnp.int32)
    cosT = cos_theta.T
    phiT = phi_theta.T
    cwin, pwin = _sc_gather(cosT, phiT, t32, n_rows, n_cols)
    m, s, amax = _tc_stream(cosT)
    loss, acc = _tc_epilogue(m, s, amax, t32.reshape(1, n_rows), cwin, pwin)
    return loss[0, 0], acc[0, 0]


# speedup vs baseline: 1.5202x; 1.0060x over previous
import functools

import jax
import jax.numpy as jnp
from jax import lax
from jax.experimental import pallas as pl
from jax.experimental.pallas import tpu as pltpu
from jax.experimental.pallas import tpu_sc as plsc

_LAMB = max(5.0, 1500.0 / 1.1)
_INV = 1.0 / (1.0 + _LAMB)
_CBLK = 2000


def _sc_gather(cosT, phiT, t32, n_rows, n_cols):
    info = plsc.get_sparse_core_info()
    nw = info.num_cores * info.num_subcores
    bpw = n_rows // nw
    mesh = plsc.VectorSubcoreMesh(core_axis_name="c", subcore_axis_name="s")

    @functools.partial(
        pl.kernel,
        mesh=mesh,
        out_type=(
            jax.ShapeDtypeStruct((n_rows, n_rows), jnp.float32),
            jax.ShapeDtypeStruct((n_rows, n_rows), jnp.float32),
        ),
        scratch_types=[
            pltpu.VMEM((bpw,), jnp.int32),
            pltpu.VMEM((bpw, n_rows), jnp.float32),
            pltpu.SemaphoreType.DMA,
        ],
    )
    def gather_kernel(cos_hbm, phi_hbm, t_hbm, cwin_out, pwin_out,
                      t_v, rows_v, sem):
        wid = lax.axis_index("s") * info.num_cores + lax.axis_index("c")
        base = wid * bpw
        pltpu.sync_copy(t_hbm.at[pl.ds(base, bpw)], t_v)
        for src, dst in ((cos_hbm, cwin_out), (phi_hbm, pwin_out)):
            pltpu.async_copy(src.at[t_v], rows_v, sem).wait()
            pltpu.sync_copy(rows_v, dst.at[pl.ds(base, bpw)])

    return gather_kernel(cosT, phiT, t32)


def _stream_body(cosT_ref, m_ref, s_ref, amax_ref):
    i = pl.program_id(0)
    x = cosT_ref[...]
    bm = jnp.max(x, axis=0, keepdims=True)
    rows = lax.broadcasted_iota(jnp.int32, x.shape, 0)
    bidx = jnp.min(jnp.where(x == bm, rows, jnp.int32(2**30)),
                   axis=0, keepdims=True) + i * _CBLK

    @pl.when(i == 0)
    def _init():
        m_ref[...] = jnp.full(bm.shape, -jnp.inf, jnp.float32)
        s_ref[...] = jnp.zeros(bm.shape, jnp.float32)
        amax_ref[...] = jnp.zeros(bm.shape, jnp.int32)

    m_old = m_ref[...]
    m_new = jnp.maximum(m_old, bm)
    bs = jnp.sum(jnp.exp(x - m_new), axis=0, keepdims=True)
    s_ref[...] = s_ref[...] * jnp.exp(m_old - m_new) + bs
    amax_ref[...] = jnp.where(bm > m_old, bidx, amax_ref[...])
    m_ref[...] = m_new


def _tc_stream(cosT):
    n_cols, n_rows = cosT.shape
    grid = (n_cols // _CBLK,)
    return pl.pallas_call(
        _stream_body,
        grid=grid,
        in_specs=[pl.BlockSpec((_CBLK, n_rows), lambda i: (i, 0))],
        out_specs=(
            pl.BlockSpec((1, n_rows), lambda i: (0, 0)),
            pl.BlockSpec((1, n_rows), lambda i: (0, 0)),
            pl.BlockSpec((1, n_rows), lambda i: (0, 0)),
        ),
        out_shape=(
            jax.ShapeDtypeStruct((1, n_rows), jnp.float32),
            jax.ShapeDtypeStruct((1, n_rows), jnp.float32),
            jax.ShapeDtypeStruct((1, n_rows), jnp.int32),
        ),
        compiler_params=pltpu.CompilerParams(
            dimension_semantics=("arbitrary",),
        ),
    )(cosT)


def _epilogue_body(m_ref, s_ref, amax_ref, trow_ref,
                   cwin_ref, pwin_ref, loss_ref, acc_ref):
    bm = m_ref[...]
    s = s_ref[...]
    amax = amax_ref[...]
    trow = trow_ref[...]
    n = trow.shape[1]
    row2 = lax.broadcasted_iota(jnp.int32, (n, n), 0)
    lane2 = lax.broadcasted_iota(jnp.int32, (n, n), 1)
    diag = row2 == lane2
    ct = jnp.sum(jnp.where(diag, cwin_ref[...], 0.0), axis=0, keepdims=True)
    pt = jnp.sum(jnp.where(diag, pwin_ref[...], 0.0), axis=0, keepdims=True)
    v = ct + (pt - ct) * _INV
    s_adj = s - jnp.exp(ct - bm) + jnp.exp(v - bm)
    logpt = v - bm - jnp.log(s_adj)
    scale = jnp.float32(1.0 / n)
    loss_ref[...] = -jnp.sum(logpt, axis=(0, 1), keepdims=True) * scale
    acc_ref[...] = jnp.sum((amax == trow).astype(jnp.float32),
                           axis=(0, 1), keepdims=True) * scale


def _tc_epilogue(m, s, amax, trow, cwin, pwin):
    n_rows = trow.shape[1]
    return pl.pallas_call(
        _epilogue_body,
        in_specs=[
            pl.BlockSpec((1, n_rows), lambda: (0, 0)),
            pl.BlockSpec((1, n_rows), lambda: (0, 0)),
            pl.BlockSpec((1, n_rows), lambda: (0, 0)),
            pl.BlockSpec((1, n_rows), lambda: (0, 0)),
            pl.BlockSpec((n_rows, n_rows), lambda: (0, 0)),
            pl.BlockSpec((n_rows, n_rows), lambda: (0, 0)),
        ],
        out_specs=(
            pl.BlockSpec((1, 1), lambda: (0, 0)),
            pl.BlockSpec((1, 1), lambda: (0, 0)),
        ),
        out_shape=(
            jax.ShapeDtypeStruct((1, 1), jnp.float32),
            jax.ShapeDtypeStruct((1, 1), jnp.float32),
        ),
    )(m, s, amax, trow, cwin, pwin)


def kernel(cos_theta, phi_theta, target):
    n_rows, n_cols = cos_theta.shape
    t32 = target.reshape(-1).astype(jnp.int32)
    cosT = cos_theta.T
    phiT = phi_theta.T
    cwin, pwin = _sc_gather(cosT, phiT, t32, n_rows, n_cols)
    m, s, amax = _tc_stream(cosT)
    loss, acc = _tc_epilogue(m, s, amax, t32.reshape(1, n_rows), cwin, pwin)
    return loss[0, 0], acc[0, 0]
